# Initial kernel scaffold; baseline (speedup 1.0000x reference)
import jax
import jax.numpy as jnp
from jax.experimental import pallas as pl


def kernel(x, edge_index, edge_weight, W1, b1, Wm0, bm0, Wm1, bm1, W2, b2, time_step):
    raise NotImplementedError("write your pallas kernel here")


# speedup vs baseline: 3.9830x; 3.9830x over previous
import functools

import jax
import jax.numpy as jnp
from jax import lax
from jax.experimental import pallas as pl
from jax.experimental.pallas import tpu as pltpu
from jax.experimental.pallas import tpu_sc as plsc

N_CORES = 2
N_SUB = 16


@functools.partial(jax.jit, static_argnames=("n", "e", "d", "c_chunk"))
def _spmm_sc(dst_idx, src_idx, edge_weight, z, *, n, e, d, c_chunk):
    nw = N_CORES * N_SUB
    e_per_tile = e // nw
    n_chunks = e_per_tile // c_chunk
    assert n_chunks * c_chunk == e_per_tile
    n_pad = 10240
    assert n <= n_pad and n_pad % (8 * N_SUB) == 0
    rows_per_tile = n_pad // N_SUB
    z_rows = 128
    n_zc = rows_per_tile // z_rows
    assert n_zc * z_rows == rows_per_tile
    lanes_per_row = d // 16

    mesh = plsc.VectorSubcoreMesh(
        core_axis_name="c", subcore_axis_name="s",
        num_cores=N_CORES, num_subcores=N_SUB,
    )

    @functools.partial(
        pl.kernel,
        out_type=jax.ShapeDtypeStruct((N_CORES, n_pad, d), jnp.float32),
        mesh=mesh,
        compiler_params=pltpu.CompilerParams(
            use_tc_tiling_on_sc=(d % 128 == 0)),
        scratch_types=[
            pltpu.VMEM((c_chunk,), jnp.int32),
            pltpu.VMEM((c_chunk,), jnp.int32),
            pltpu.VMEM((c_chunk,), jnp.float32),
            pltpu.VMEM((c_chunk, d), jnp.float32),
            pltpu.VMEM((z_rows, d), jnp.float32),
            pltpu.VMEM_SHARED((n_pad, d), jnp.float32),
            pltpu.SemaphoreType.DMA,
        ],
    )
    def spmm(dsti_hbm, srci_hbm, w_hbm, z_hbm, out_hbm,
             src_v, dst_v, w_v, rows_v, stage_v, acc_sh, sem):
        c = lax.axis_index("c")
        s = lax.axis_index("s")
        row0 = s * rows_per_tile

        zero16 = jnp.zeros((16,), jnp.float32)

        def zero_row(i, carry):
            for j in range(lanes_per_row):
                stage_v[i, pl.ds(j * 16, 16)] = zero16
            return carry

        lax.fori_loop(0, z_rows, zero_row, 0)

        def zero_acc(k, carry):
            pltpu.sync_copy(
                stage_v, acc_sh.at[pl.ds(row0 + k * z_rows, z_rows)])
            return carry

        lax.fori_loop(0, n_zc, zero_acc, 0)
        plsc.subcore_barrier()

        ebase = (c * N_SUB + s) * e_per_tile

        def edge_chunk(i, carry):
            b = ebase + i * c_chunk
            pltpu.sync_copy(srci_hbm.at[pl.ds(b, c_chunk)], src_v)
            pltpu.sync_copy(dsti_hbm.at[pl.ds(b, c_chunk)], dst_v)
            pltpu.sync_copy(w_hbm.at[pl.ds(b, c_chunk)], w_v)
            pltpu.async_copy(z_hbm.at[src_v], rows_v, sem).wait()

            def scale(g, carry2):
                wvec = w_v[pl.ds(g * 16, 16)]
                dn = lax.GatherDimensionNumbers(
                    offset_dims=(), collapsed_slice_dims=(0,),
                    start_index_map=(0,))
                for k in range(16):
                    wv = lax.gather(
                        wvec, jnp.full((16, 1), k, jnp.int32), dn,
                        slice_sizes=(1,),
                        mode=lax.GatherScatterMode.PROMISE_IN_BOUNDS)
                    ee = g * 16 + k
                    for j in range(lanes_per_row):
                        sl = pl.ds(j * 16, 16)
                        rows_v[ee, sl] = rows_v[ee, sl] * wv
                return carry2

            lax.fori_loop(0, c_chunk // 16, scale, 0)
            pltpu.sync_copy(rows_v, acc_sh.at[dst_v], add=True)
            return carry

        lax.fori_loop(0, n_chunks, edge_chunk, 0)
        plsc.subcore_barrier()

        def out_copy(k, carry):
            r = row0 + k * z_rows
            pltpu.sync_copy(acc_sh.at[pl.ds(r, z_rows)], stage_v)
            pltpu.sync_copy(stage_v, out_hbm.at[c, pl.ds(r, z_rows)])
            return carry

        lax.fori_loop(0, n_zc, out_copy, 0)

    return spmm(dst_idx, src_idx, edge_weight, z)


def _mm(x, wt, b, blk=1000):
    n, d_in = x.shape
    d_out = wt.shape[1]

    def kern(x_ref, wt_ref, b_ref, o_ref):
        o_ref[...] = jnp.dot(
            x_ref[...], wt_ref[...],
            preferred_element_type=jnp.float32) + b_ref[...]

    return pl.pallas_call(
        kern,
        grid=(n // blk,),
        in_specs=[
            pl.BlockSpec((blk, d_in), lambda i: (i, 0)),
            pl.BlockSpec((d_in, d_out), lambda i: (0, 0)),
            pl.BlockSpec((1, d_out), lambda i: (0, 0)),
        ],
        out_specs=pl.BlockSpec((blk, d_out), lambda i: (i, 0)),
        out_shape=jax.ShapeDtypeStruct((n, d_out), jnp.float32),
    )(x, wt, b)


def _fuse(p, h_prev, dt, wt, b, *, first, want_h, blk=1000):
    _, n, d = p.shape
    d_out = wt.shape[1]

    def kern(*refs):
        if first:
            p_ref, dt_ref, wt_ref, b_ref = refs[:4]
            outs = refs[4:]
        else:
            p_ref, h_ref, dt_ref, wt_ref, b_ref = refs[:5]
            outs = refs[5:]
        f = jnp.maximum(p_ref[0] + p_ref[1], 0.0)
        if first:
            h = f
        else:
            h = h_ref[...] + dt_ref[0, 0] * f
        z = jnp.dot(h, wt_ref[...],
                    preferred_element_type=jnp.float32) + b_ref[...]
        if want_h:
            outs[0][...] = h
            outs[1][...] = z
        else:
            outs[0][...] = z

    in_specs = [pl.BlockSpec((N_CORES, blk, d), lambda i: (0, i, 0))]
    args = [p]
    if not first:
        in_specs.append(pl.BlockSpec((blk, d), lambda i: (i, 0)))
        args.append(h_prev)
    in_specs += [
        pl.BlockSpec((1, 128), lambda i: (0, 0)),
        pl.BlockSpec((d, d_out), lambda i: (0, 0)),
        pl.BlockSpec((1, d_out), lambda i: (0, 0)),
    ]
    args += [dt, wt, b]

    z_spec = pl.BlockSpec((blk, d_out), lambda i: (i, 0))
    z_shape = jax.ShapeDtypeStruct((n, d_out), jnp.float32)
    if want_h:
        out_specs = [pl.BlockSpec((blk, d), lambda i: (i, 0)), z_spec]
        out_shape = [jax.ShapeDtypeStruct((n, d), jnp.float32), z_shape]
    else:
        out_specs = [z_spec]
        out_shape = [z_shape]

    res = pl.pallas_call(
        kern,
        grid=(n // blk,),
        in_specs=in_specs,
        out_specs=out_specs,
        out_shape=out_shape,
    )(*args)
    return res if want_h else res[0]


def _final_add(p):
    _, m, d = p.shape

    def kern(p_ref, o_ref):
        o_ref[...] = p_ref[0] + p_ref[1]

    return pl.pallas_call(
        kern,
        grid=(1,),
        in_specs=[pl.BlockSpec((N_CORES, m, d), lambda i: (0, 0, 0))],
        out_specs=pl.BlockSpec((m, d), lambda i: (0, 0)),
        out_shape=jax.ShapeDtypeStruct((m, d), jnp.float32),
    )(p)


def kernel(x, edge_index, edge_weight, W1, b1, Wm0, bm0, Wm1, bm1, W2, b2,
           time_step):
    n, d_in = x.shape
    e = edge_index.shape[1]
    d_h = W1.shape[0]
    n_cls = W2.shape[0]

    dt = jnp.full((1, 128), time_step[0], dtype=jnp.float32)
    dst_idx = edge_index[0]
    src_idx = edge_index[1]

    z1 = _mm(x, W1.T, b1.reshape(1, -1))
    p1 = _spmm_sc(dst_idx, src_idx, edge_weight, z1, n=n, e=e, d=d_h, c_chunk=80)
    h1, z2 = _fuse(p1, None, dt, Wm0.T, bm0.reshape(1, -1),
                   first=True, want_h=True)
    p2 = _spmm_sc(dst_idx, src_idx, edge_weight, z2, n=n, e=e, d=d_h, c_chunk=80)
    h2, z3 = _fuse(p2, h1, dt, Wm1.T, bm1.reshape(1, -1),
                   first=False, want_h=True)
    p3 = _spmm_sc(dst_idx, src_idx, edge_weight, z3, n=n, e=e, d=d_h, c_chunk=80)
    z4 = _fuse(p3, h2, dt, W2.T, b2.reshape(1, -1),
               first=False, want_h=False)
    p4 = _spmm_sc(dst_idx, src_idx, edge_weight, z4, n=n, e=e, d=n_cls, c_chunk=80)
    n_pad = p4.shape[1]
    out = _final_add(p4.reshape(N_CORES, (n_pad * n_cls) // 128, 128))
    return out.reshape(n_pad, n_cls)[:n]
